# baseline (speedup 1.0000x reference)
import functools

import jax
import jax.numpy as jnp
from jax import lax
from jax.experimental import pallas as pl
from jax.experimental.pallas import tpu as pltpu
from jax.experimental.pallas import tpu_sc as plsc

BATCH = 16384
DIM = 64
CHUNK = 128


@functools.lru_cache(maxsize=None)
def _build():
    info = plsc.get_sparse_core_info()
    nc, ns = info.num_cores, info.num_subcores
    nw = nc * ns
    b_per_w = BATCH // nw
    kch = b_per_w // CHUNK

    mesh = plsc.VectorSubcoreMesh(core_axis_name="c", subcore_axis_name="s")

    @functools.partial(
        pl.kernel,
        mesh=mesh,
        out_type=(
            jax.ShapeDtypeStruct((BATCH, DIM), jnp.float32),
            jax.ShapeDtypeStruct((BATCH, DIM), jnp.float32),
        ),
        scratch_types=[
            pltpu.VMEM((kch, CHUNK), jnp.int32),
            pltpu.VMEM((kch, CHUNK), jnp.int32),
            pltpu.VMEM((b_per_w, DIM), jnp.float32),
            pltpu.VMEM((b_per_w, DIM), jnp.float32),
            pltpu.SemaphoreType.DMA,
            pltpu.SemaphoreType.DMA,
        ],
        compiler_params=pltpu.CompilerParams(use_tc_tiling_on_sc=False),
    )
    def emb_kernel(user_ids, item_ids, emb_user, emb_item, out_u, out_i,
                   uidx_v, iidx_v, urows_v, irows_v, usem, isem):
        wid = lax.axis_index("s") * nc + lax.axis_index("c")
        base = wid * b_per_w
        pltpu.sync_copy(user_ids.at[wid], uidx_v)
        pltpu.sync_copy(item_ids.at[wid], iidx_v)
        ucps = [
            pltpu.async_copy(
                emb_user.at[uidx_v.at[j]],
                urows_v.at[pl.ds(j * CHUNK, CHUNK)],
                usem,
            )
            for j in range(kch)
        ]
        icps = [
            pltpu.async_copy(
                emb_item.at[iidx_v.at[j]],
                irows_v.at[pl.ds(j * CHUNK, CHUNK)],
                isem,
            )
            for j in range(kch)
        ]
        for cp in ucps:
            cp.wait()
        pltpu.sync_copy(urows_v, out_u.at[pl.ds(base, b_per_w)])
        for cp in icps:
            cp.wait()
        pltpu.sync_copy(irows_v, out_i.at[pl.ds(base, b_per_w)])

    return emb_kernel, nw, kch


def kernel(user_ids, item_ids, emb_user, emb_item):
    emb_kernel, nw, kch = _build()
    uid3 = user_ids.reshape(nw, kch, CHUNK)
    iid3 = item_ids.reshape(nw, kch, CHUNK)
    return emb_kernel(uid3, iid3, emb_user, emb_item)
